# baseline (speedup 1.0000x reference)
import functools

import jax
import jax.numpy as jnp
from jax import lax
from jax.experimental import pallas as pl
from jax.experimental.pallas import tpu as pltpu
from jax.experimental.pallas import tpu_sc as plsc

VOCAB = 100000
NGRAM_VOCAB = 1000000
DIM = 64
BATCH = 4096
NGRAMS = 20

NC, NS, L = 2, 16, 16
NW = NC * NS
BW = BATCH // NW
DV = DIM // L


def _body(we, ne, cwi, cni, xwi, xni, out,
          widx, raw, tn, wbuf, rows, cacc, xacc, outv,
          sem_w, sem0, sem1):
    wid = lax.axis_index("s") * NC + lax.axis_index("c")
    base = wid * BW
    iota = jax.lax.iota(jnp.int32, L)
    sems = (sem0, sem1)

    def one_side(word_idx_hbm, ngram_idx_hbm, acc):
        pltpu.sync_copy(word_idx_hbm.at[pl.ds(base, BW)], widx)
        pltpu.sync_copy(ngram_idx_hbm.at[pl.ds(base * NGRAMS, BW * NGRAMS)], raw)

        cp_w = pltpu.async_copy(we.at[widx], wbuf, sem_w)

        @pl.loop(0, NGRAMS)
        def _(g):
            for b0 in range(BW // L):
                v = plsc.load_gather(raw, [(b0 * L + iota) * NGRAMS + g])
                tn[g, pl.ds(b0 * L, L)] = v

        cps = [None, None]
        cps[0] = pltpu.async_copy(ne.at[tn.at[0]], rows.at[0], sems[0])

        cp_w.wait()

        @pl.loop(0, BW)
        def _(r):
            for d in range(DV):
                acc[pl.ds(r * DIM + d * L, L)] = wbuf[r, pl.ds(d * L, L)]

        for g in range(NGRAMS):
            if g + 1 < NGRAMS:
                cps[(g + 1) % 2] = pltpu.async_copy(
                    ne.at[tn.at[g + 1]], rows.at[(g + 1) % 2], sems[(g + 1) % 2])
            cps[g % 2].wait()
            buf = rows.at[g % 2]

            @pl.loop(0, BW)
            def _(r):
                for d in range(DV):
                    acc[pl.ds(r * DIM + d * L, L)] = (
                        acc[pl.ds(r * DIM + d * L, L)] + buf[r, pl.ds(d * L, L)])

    one_side(cwi, cni, cacc)
    one_side(xwi, xni, xacc)

    for b0 in range(BW // L):
        ridx = (b0 * L + iota) * DIM

        @pl.loop(0, DIM, init_carry=jnp.zeros((L,), jnp.float32))
        def s(d, s):
            c = plsc.load_gather(cacc, [ridx + d])
            x = plsc.load_gather(xacc, [ridx + d])
            return s + c * x

        outv[pl.ds(b0 * L, L)] = s

    pltpu.sync_copy(outv, out.at[pl.ds(base, BW)])


@jax.jit
def _run(we, ne, cwi, cni, xwi, xni):
    mesh = plsc.VectorSubcoreMesh(core_axis_name="c", subcore_axis_name="s",
                                  num_cores=NC, num_subcores=NS)
    f = pl.kernel(
        _body,
        out_type=jax.ShapeDtypeStruct((BATCH,), jnp.float32),
        mesh=mesh,
        compiler_params=pltpu.CompilerParams(
            needs_layout_passes=False, use_tc_tiling_on_sc=False),
        scratch_types=[
            pltpu.VMEM((BW,), jnp.int32),
            pltpu.VMEM((BW * NGRAMS,), jnp.int32),
            pltpu.VMEM((NGRAMS, BW), jnp.int32),
            pltpu.VMEM((BW, DIM), jnp.float32),
            pltpu.VMEM((2, BW, DIM), jnp.float32),
            pltpu.VMEM((BW * DIM,), jnp.float32),
            pltpu.VMEM((BW * DIM,), jnp.float32),
            pltpu.VMEM((BW,), jnp.float32),
            pltpu.SemaphoreType.DMA,
            pltpu.SemaphoreType.DMA,
            pltpu.SemaphoreType.DMA,
        ],
    )
    return f(we, ne, cwi, cni, xwi, xni)


def kernel(word_embeddings, ngram_embeddings, center_word_idx,
           center_ngram_idxs, context_word_idx, context_ngram_idxs):
    return _run(
        word_embeddings, ngram_embeddings,
        center_word_idx.astype(jnp.int32),
        center_ngram_idxs.astype(jnp.int32).reshape(-1),
        context_word_idx.astype(jnp.int32),
        context_ngram_idxs.astype(jnp.int32).reshape(-1))


# speedup vs baseline: 1.0147x; 1.0147x over previous
import jax
import jax.numpy as jnp
from jax import lax
from jax.experimental import pallas as pl
from jax.experimental.pallas import tpu as pltpu
from jax.experimental.pallas import tpu_sc as plsc

VOCAB = 100000
NGRAM_VOCAB = 1000000
DIM = 64
BATCH = 4096
NGRAMS = 20

NC, NS, L = 2, 16, 16
NW = NC * NS
BW = BATCH // NW
DV = DIM // L
NBUF = 4


def _body(we, ne, cwi, cni, xwi, xni, out,
          widx, raw, tn, tgt, wbuf, rows, cacc, xacc, outv, shacc,
          sem_w0, sem_w1, gs0, gs1, gs2, gs3, ss0, ss1, ss2, ss3):
    cid = lax.axis_index("c")
    sid = lax.axis_index("s")
    wid = sid * NC + cid
    base = wid * BW
    iota = jax.lax.iota(jnp.int32, L)
    gsems = (gs0, gs1, gs2, gs3)
    ssems = (ss0, ss1, ss2, ss3)
    wsems = (sem_w0, sem_w1)

    word_cps = []
    for k, wsrc in enumerate((cwi, xwi)):
        pltpu.sync_copy(wsrc.at[pl.ds(base, BW)], widx.at[k])
        word_cps.append(
            pltpu.async_copy(we.at[widx.at[k]], wbuf.at[k], wsems[k]))

    for k in range(2):
        rowbase = (sid * 2 + k) * BW
        for i in range(BW // L):
            tgt[k, pl.ds(i * L, L)] = rowbase + i * L + iota

    for k, nsrc in enumerate((cni, xni)):
        pltpu.sync_copy(nsrc.at[pl.ds(base * NGRAMS, BW * NGRAMS)], raw)

        @pl.loop(0, NGRAMS)
        def _(g):
            for b0 in range(BW // L):
                v = plsc.load_gather(raw, [(b0 * L + iota) * NGRAMS + g])
                tn[k, g, pl.ds(b0 * L, L)] = v

    for k in range(2):
        word_cps[k].wait()
        pltpu.sync_copy(wbuf.at[k], shacc.at[tgt.at[k]])

    steps = [(k, g) for g in range(NGRAMS) for k in range(2)]
    gather_cp = [None] * NBUF
    sct_cp = [None] * NBUF

    def fire_gather(step_i):
        k, g = steps[step_i]
        j = step_i % NBUF
        gather_cp[j] = pltpu.async_copy(
            ne.at[tn.at[k, g]], rows.at[j], gsems[j])

    for i in range(NBUF):
        fire_gather(i)

    for i in range(len(steps)):
        k, g = steps[i]
        j = i % NBUF
        gather_cp[j].wait()
        sct_cp[j] = pltpu.async_copy(
            rows.at[j], shacc.at[tgt.at[k]], ssems[j], add=True)
        if i + NBUF < len(steps):
            sct_cp[j].wait()
            sct_cp[j] = None
            fire_gather(i + NBUF)

    for j in range(NBUF):
        if sct_cp[j] is not None:
            sct_cp[j].wait()

    pltpu.sync_copy(shacc.at[pl.ds((sid * 2) * BW, BW), :], cacc)
    pltpu.sync_copy(shacc.at[pl.ds((sid * 2 + 1) * BW, BW), :], xacc)

    for b0 in range(BW // L):
        ridx = b0 * L + iota

        @pl.loop(0, DIM, init_carry=jnp.zeros((L,), jnp.float32))
        def s(d, s):
            col = jnp.full((L,), 0, jnp.int32) + d
            c = plsc.load_gather(cacc, [ridx, col])
            x = plsc.load_gather(xacc, [ridx, col])
            return s + c * x

        outv[pl.ds(b0 * L, L)] = s

    pltpu.sync_copy(outv, out.at[pl.ds(base, BW)])


@jax.jit
def _run(we, ne, cwi, cni, xwi, xni):
    mesh = plsc.VectorSubcoreMesh(core_axis_name="c", subcore_axis_name="s",
                                  num_cores=NC, num_subcores=NS)
    f = pl.kernel(
        _body,
        out_type=jax.ShapeDtypeStruct((BATCH,), jnp.float32),
        mesh=mesh,
        compiler_params=pltpu.CompilerParams(
            needs_layout_passes=False, use_tc_tiling_on_sc=False),
        scratch_types=[
            pltpu.VMEM((2, BW), jnp.int32),
            pltpu.VMEM((BW * NGRAMS,), jnp.int32),
            pltpu.VMEM((2, NGRAMS, BW), jnp.int32),
            pltpu.VMEM((2, BW), jnp.int32),
            pltpu.VMEM((2, BW, DIM), jnp.float32),
            pltpu.VMEM((NBUF, BW, DIM), jnp.float32),
            pltpu.VMEM((BW, DIM), jnp.float32),
            pltpu.VMEM((BW, DIM), jnp.float32),
            pltpu.VMEM((BW,), jnp.float32),
            pltpu.VMEM_SHARED((NS * 2 * BW, DIM), jnp.float32),
            pltpu.SemaphoreType.DMA,
            pltpu.SemaphoreType.DMA,
            pltpu.SemaphoreType.DMA,
            pltpu.SemaphoreType.DMA,
            pltpu.SemaphoreType.DMA,
            pltpu.SemaphoreType.DMA,
            pltpu.SemaphoreType.DMA,
            pltpu.SemaphoreType.DMA,
            pltpu.SemaphoreType.DMA,
            pltpu.SemaphoreType.DMA,
        ],
    )
    return f(we, ne, cwi, cni, xwi, xni)


def kernel(word_embeddings, ngram_embeddings, center_word_idx,
           center_ngram_idxs, context_word_idx, context_ngram_idxs):
    return _run(
        word_embeddings, ngram_embeddings,
        center_word_idx.astype(jnp.int32),
        center_ngram_idxs.astype(jnp.int32).reshape(-1),
        context_word_idx.astype(jnp.int32),
        context_ngram_idxs.astype(jnp.int32).reshape(-1))
